# baseline (speedup 1.0000x reference)
import functools

import jax
import jax.numpy as jnp
from jax import lax
from jax.experimental import pallas as pl
from jax.experimental.pallas import tpu as pltpu
from jax.experimental.pallas import tpu_sc as plsc

_N, _E, _H, _F = 50000, 800000, 16, 4
_NC, _NS = 2, 16
_NW = _NC * _NS
_ECH = 1000
_R8C = _ECH // 8
_NCHTOT = _E // _ECH
_NCHUNK = _NCHTOT // _NW
_NPS = _N // _NS

def _make_gather_body(chunk0, ncw):
  def _gather_body(h_loc, a_src, a_dst, out_src8, out_dst8,
                   idx_s, idx_d, rows_s, rows_d, sem):
    w = lax.axis_index("s") * _NC + lax.axis_index("c")

    def chunk(i, carry):
        lc = w * ncw + i
        c = chunk0 + lc
        r8 = lc * _R8C
        h1 = pltpu.async_copy(a_src.at[pl.ds(c * 8, 8)], idx_s, sem)
        h2 = pltpu.async_copy(a_dst.at[pl.ds(c * 8, 8)], idx_d, sem)
        h1.wait()
        h2.wait()
        handles = []
        for k in range(8):
            handles.append(pltpu.async_copy(
                h_loc.at[idx_s.at[k]],
                rows_s.at[pl.ds(k * _R8C, _R8C)], sem))
            handles.append(pltpu.async_copy(
                h_loc.at[idx_d.at[k]],
                rows_d.at[pl.ds(k * _R8C, _R8C)], sem))
        for h in handles:
            h.wait()
        handles = []
        for k in range(8):
            handles.append(pltpu.async_copy(
                rows_s.at[pl.ds(k * _R8C, _R8C)],
                out_src8.at[pl.ds(r8, _R8C), pl.ds(k * _H, _H)], sem))
            handles.append(pltpu.async_copy(
                rows_d.at[pl.ds(k * _R8C, _R8C)],
                out_dst8.at[pl.ds(r8, _R8C), pl.ds(k * _H, _H)], sem))
        for h in handles:
            h.wait()
        return carry

    lax.fori_loop(0, ncw, chunk, 0)
  return _gather_body


@functools.cache
def _gather(chunk0, ncw):
    rows = ncw * _NW * _R8C
    mesh = plsc.VectorSubcoreMesh(core_axis_name="c", subcore_axis_name="s")
    return pl.kernel(
        _make_gather_body(chunk0, ncw),
        out_type=(jax.ShapeDtypeStruct((rows, 128), jnp.float32),
                  jax.ShapeDtypeStruct((rows, 128), jnp.float32)),
        mesh=mesh,
        scratch_types=[
            pltpu.VMEM((8, _R8C), jnp.int32),
            pltpu.VMEM((8, _R8C), jnp.int32),
            pltpu.VMEM((_ECH, _H), jnp.float32),
            pltpu.VMEM((_ECH, _H), jnp.float32),
            pltpu.SemaphoreType.DMA,
        ],
        compiler_params=pltpu.CompilerParams(use_tc_tiling_on_sc=False),
    )


def _make_scatter_body(chunk0, ncw):
  def _scatter_body(zeros_h, a_src, a_dst, d_src8, d_dst8, out,
                    acc_sh, idx_s, idx_d, del_s, del_d, sem):
    c = lax.axis_index("c")
    s = lax.axis_index("s")
    w = s * _NC + c
    pltpu.sync_copy(zeros_h.at[pl.ds(s * _NPS, _NPS)],
                    acc_sh.at[pl.ds(s * _NPS, _NPS)])
    plsc.subcore_barrier()

    def chunk(i, carry):
        lc = w * ncw + i
        ch = chunk0 + lc
        r8 = lc * _R8C
        h1 = pltpu.async_copy(a_src.at[pl.ds(ch * 8, 8)], idx_s, sem)
        h2 = pltpu.async_copy(a_dst.at[pl.ds(ch * 8, 8)], idx_d, sem)
        handles = []
        for k in range(8):
            handles.append(pltpu.async_copy(
                d_src8.at[pl.ds(r8, _R8C), pl.ds(k * _H, _H)],
                del_s.at[pl.ds(k * _R8C, _R8C)], sem))
            handles.append(pltpu.async_copy(
                d_dst8.at[pl.ds(r8, _R8C), pl.ds(k * _H, _H)],
                del_d.at[pl.ds(k * _R8C, _R8C)], sem))
        h1.wait()
        h2.wait()
        for h in handles:
            h.wait()
        handles = []
        for k in range(8):
            handles.append(pltpu.async_copy(
                del_s.at[pl.ds(k * _R8C, _R8C)],
                acc_sh.at[idx_s.at[k]], sem, add=True))
            handles.append(pltpu.async_copy(
                del_d.at[pl.ds(k * _R8C, _R8C)],
                acc_sh.at[idx_d.at[k]], sem, add=True))
        for h in handles:
            h.wait()
        return carry

    lax.fori_loop(0, ncw, chunk, 0)
    plsc.subcore_barrier()
    pltpu.sync_copy(acc_sh.at[pl.ds(s * _NPS, _NPS)],
                    out.at[c, pl.ds(s * _NPS, _NPS)])
  return _scatter_body


@functools.cache
def _scatter(chunk0, ncw):
    mesh = plsc.VectorSubcoreMesh(core_axis_name="c", subcore_axis_name="s")
    return pl.kernel(
        _make_scatter_body(chunk0, ncw),
        out_type=jax.ShapeDtypeStruct((_NC, _N, _H), jnp.float32),
        mesh=mesh,
        scratch_types=[
            pltpu.VMEM_SHARED((_N, _H), jnp.float32),
            pltpu.VMEM((8, _R8C), jnp.int32),
            pltpu.VMEM((8, _R8C), jnp.int32),
            pltpu.VMEM((_ECH, _H), jnp.float32),
            pltpu.VMEM((_ECH, _H), jnp.float32),
            pltpu.SemaphoreType.DMA,
        ],
        compiler_params=pltpu.CompilerParams(use_tc_tiling_on_sc=False),
    )

_EB = 16000
_RB = _EB // 8
_R8 = _E // 8


def _mlp_body(hs_ref, hd_ref, ef_ref, g_ref, w1g_ref, b1_ref,
              arep_ref, brep_ref, crep_ref, w2rep_ref, b2rep_ref,
              os_ref, od_ref):
    f32 = jnp.float32
    hs = hs_ref[...]
    hd = hd_ref[...]
    ef = ef_ref[...]
    g = g_ref[...]
    hs = jnp.where(jnp.isnan(hs), 0.0, hs)
    hd = jnp.where(jnp.isnan(hd), 0.0, hd)
    ef = jnp.where(jnp.isnan(ef), 0.0, ef)
    g = jnp.where(jnp.isnan(g), 0.0, g)
    const_row = jnp.dot(g, w1g_ref[...], preferred_element_type=f32) \
        + b1_ref[...]
    const_rep = jnp.concatenate([const_row] * 8, axis=1)
    hid = (jnp.dot(hs, arep_ref[...], preferred_element_type=f32)
           + jnp.dot(hd, brep_ref[...], preferred_element_type=f32)
           + jnp.dot(ef, crep_ref[...], preferred_element_type=f32)
           + const_rep)
    hid = jnp.tanh(hid)
    out = jnp.dot(hid, w2rep_ref[...], preferred_element_type=f32) \
        + b2rep_ref[...]
    out = jnp.tanh(out)
    os_ref[...] = out[:, 0:128]
    od_ref[...] = out[:, 128:256]


def _mlp(hs8, hd8, ef8, g, w1g, b1, arep, brep, crep, w2rep, b2rep):
    rows = hs8.shape[0]
    full = lambda shape: pl.BlockSpec(shape, lambda i: (0, 0))
    return pl.pallas_call(
        _mlp_body,
        grid=(rows // _RB,),
        in_specs=[
            pl.BlockSpec((_RB, 128), lambda i: (i, 0)),
            pl.BlockSpec((_RB, 128), lambda i: (i, 0)),
            pl.BlockSpec((_RB, 32), lambda i: (i, 0)),
            full((1, 11)),
            full((11, 64)),
            full((1, 64)),
            full((128, 512)),
            full((128, 512)),
            full((32, 512)),
            full((512, 256)),
            full((1, 256)),
        ],
        out_specs=[
            pl.BlockSpec((_RB, 128), lambda i: (i, 0)),
            pl.BlockSpec((_RB, 128), lambda i: (i, 0)),
        ],
        out_shape=[
            jax.ShapeDtypeStruct((rows, 128), jnp.float32),
            jax.ShapeDtypeStruct((rows, 128), jnp.float32),
        ],
    )(hs8, hd8, ef8, g, w1g, b1, arep, brep, crep, w2rep, b2rep)


def _block_diag(w, reps):
    k, m = w.shape
    out = jnp.zeros((reps * k, reps * m), w.dtype)
    for i in range(reps):
        out = out.at[i * k:(i + 1) * k, i * m:(i + 1) * m].set(w)
    return out


_NB = 5000


def _combine_body(a_ref, o_ref):
    a = a_ref[...]
    acc = a[0]
    for j in range(1, a.shape[0]):
        acc = acc + a[j]
    o_ref[...] = jnp.tanh(acc)


def _combine(acc):
    nparts = acc.shape[0]
    return pl.pallas_call(
        _combine_body,
        grid=(_N // _NB,),
        in_specs=[pl.BlockSpec((nparts, _NB, _H), lambda i: (0, i, 0))],
        out_specs=pl.BlockSpec((_NB, _H), lambda i: (i, 0)),
        out_shape=jax.ShapeDtypeStruct((_N, _H), jnp.float32),
    )(acc)


def kernel(h_local, h_global, addr_src, addr_dst, edge_feats, g_feat, t,
           W1_src, b1_src, W2_src, b2_src, W1_dst, b1_dst, W2_dst, b2_dst):
    a_src2 = addr_src.reshape(_NCHTOT * 8, _R8C)
    a_dst2 = addr_dst.reshape(_NCHTOT * 8, _R8C)

    gvec = jnp.concatenate([h_global, g_feat, t], axis=1)
    w1 = jnp.concatenate([W1_src, W1_dst], axis=1)
    b1 = jnp.concatenate([b1_src, b1_dst])[None, :]
    w1g = w1[2 * _H + _F:, :]
    arep = _block_diag(w1[0:_H, :], 8)
    brep = _block_diag(w1[_H:2 * _H, :], 8)
    crep = _block_diag(w1[2 * _H:2 * _H + _F, :], 8)
    w2rep = jnp.zeros((512, 256), jnp.float32)
    for e in range(8):
        w2rep = w2rep.at[e * 64:e * 64 + 32, e * 16:e * 16 + 16].set(W2_src)
        w2rep = w2rep.at[e * 64 + 32:e * 64 + 64,
                         128 + e * 16:128 + e * 16 + 16].set(W2_dst)
    b2rep = jnp.concatenate([jnp.tile(b2_src, 8), jnp.tile(b2_dst, 8)])[None, :]

    ef_perm = edge_feats.reshape(_NCHTOT, 8, _R8C, _F) \
        .transpose(0, 2, 1, 3).reshape(_R8, 8 * _F)
    zeros = jnp.zeros((_N, _H), jnp.float32)

    accs = []
    for chunk0, ncw in ((0, 13), (13 * _NW, 12)):
        r0 = chunk0 * _R8C
        nrows = ncw * _NW * _R8C
        hs, hd = _gather(chunk0, ncw)(h_local, a_src2, a_dst2)
        d_s, d_d = _mlp(hs, hd, ef_perm[r0:r0 + nrows], gvec,
                        w1g, b1, arep, brep, crep, w2rep, b2rep)
        accs.append(_scatter(chunk0, ncw)(zeros, a_src2, a_dst2, d_s, d_d))
    return _combine(jnp.concatenate(accs, axis=0))


# speedup vs baseline: 1.0556x; 1.0556x over previous
import functools

import jax
import jax.numpy as jnp
from jax import lax
from jax.experimental import pallas as pl
from jax.experimental.pallas import tpu as pltpu
from jax.experimental.pallas import tpu_sc as plsc

_N, _E, _H, _F = 50000, 800000, 16, 4
_NC, _NS = 2, 16
_NW = _NC * _NS
_BND = 125
_ECH = 5000
_NBAND = _ECH // _BND
_RCH = _ECH // 8
_R8C = 125
_NCHTOT = _E // 1000
_NCH5 = _E // _ECH
_NCHUNK = _NCH5 // _NW
_NCHUNK1 = _NCHTOT // _NW
_NPS = _N // _NS

def _gather_body(h_loc, a_src, a_dst, out_src8, out_dst8,
                 idx_s, idx_d, rows, sem):
    w = lax.axis_index("s") * _NC + lax.axis_index("c")

    def one_list(a_ref, idx, out8, r8):
        handles = []
        for k in range(_NBAND):
            handles.append(pltpu.async_copy(
                h_loc.at[idx.at[k]],
                rows.at[pl.ds(k * _BND, _BND)], sem))
        for h in handles:
            h.wait()
        handles = []
        for k in range(_NBAND):
            j, m = divmod(k, 8)
            handles.append(pltpu.async_copy(
                rows.at[pl.ds(k * _BND, _BND)],
                out8.at[pl.ds(r8 + j * _R8C, _R8C), pl.ds(m * _H, _H)],
                sem))
        for h in handles:
            h.wait()

    def chunk(i, carry):
        c = w * _NCHUNK + i
        r8 = c * _RCH
        h1 = pltpu.async_copy(a_src.at[pl.ds(c * _NBAND, _NBAND)], idx_s, sem)
        h2 = pltpu.async_copy(a_dst.at[pl.ds(c * _NBAND, _NBAND)], idx_d, sem)
        h1.wait()
        h2.wait()
        one_list(a_src, idx_s, out_src8, r8)
        one_list(a_dst, idx_d, out_dst8, r8)
        return carry

    lax.fori_loop(0, _NCHUNK, chunk, 0)


@functools.cache
def _gather():
    mesh = plsc.VectorSubcoreMesh(core_axis_name="c", subcore_axis_name="s")
    return pl.kernel(
        _gather_body,
        out_type=(jax.ShapeDtypeStruct((_E // 8, 128), jnp.float32),
                  jax.ShapeDtypeStruct((_E // 8, 128), jnp.float32)),
        mesh=mesh,
        scratch_types=[
            pltpu.VMEM((_NBAND, _BND), jnp.int32),
            pltpu.VMEM((_NBAND, _BND), jnp.int32),
            pltpu.VMEM((_ECH, _H), jnp.float32),
            pltpu.SemaphoreType.DMA,
        ],
        compiler_params=pltpu.CompilerParams(use_tc_tiling_on_sc=False),
    )


def _scatter_body(zeros_h, a_src, a_dst, d_src8, d_dst8, out,
                  acc_sh, idx_s, idx_d, dels, sem):
    c = lax.axis_index("c")
    s = lax.axis_index("s")
    w = s * _NC + c
    pltpu.sync_copy(zeros_h.at[pl.ds(s * _NPS, _NPS)],
                    acc_sh.at[pl.ds(s * _NPS, _NPS)])
    plsc.subcore_barrier()

    def chunk(i, carry):
        ch = w * _NCHUNK1 + i
        r8 = ch * _R8C
        h1 = pltpu.async_copy(a_src.at[pl.ds(ch * 8, 8)], idx_s, sem)
        h2 = pltpu.async_copy(a_dst.at[pl.ds(ch * 8, 8)], idx_d, sem)
        handles = []
        for k in range(8):
            handles.append(pltpu.async_copy(
                d_src8.at[pl.ds(r8, _R8C), pl.ds(k * _H, _H)],
                dels.at[pl.ds(k * _BND, _BND)], sem))
            handles.append(pltpu.async_copy(
                d_dst8.at[pl.ds(r8, _R8C), pl.ds(k * _H, _H)],
                dels.at[pl.ds((8 + k) * _BND, _BND)], sem))
        h1.wait()
        h2.wait()
        for h in handles:
            h.wait()
        handles = []
        for k in range(8):
            handles.append(pltpu.async_copy(
                dels.at[pl.ds(k * _BND, _BND)],
                acc_sh.at[idx_s.at[k]], sem, add=True))
            handles.append(pltpu.async_copy(
                dels.at[pl.ds((8 + k) * _BND, _BND)],
                acc_sh.at[idx_d.at[k]], sem, add=True))
        for h in handles:
            h.wait()
        return carry

    lax.fori_loop(0, _NCHUNK1, chunk, 0)
    plsc.subcore_barrier()
    pltpu.sync_copy(acc_sh.at[pl.ds(s * _NPS, _NPS)],
                    out.at[c, pl.ds(s * _NPS, _NPS)])


@functools.cache
def _scatter():
    mesh = plsc.VectorSubcoreMesh(core_axis_name="c", subcore_axis_name="s")
    return pl.kernel(
        _scatter_body,
        out_type=jax.ShapeDtypeStruct((_NC, _N, _H), jnp.float32),
        mesh=mesh,
        scratch_types=[
            pltpu.VMEM_SHARED((_N, _H), jnp.float32),
            pltpu.VMEM((8, _BND), jnp.int32),
            pltpu.VMEM((8, _BND), jnp.int32),
            pltpu.VMEM((2000, _H), jnp.float32),
            pltpu.SemaphoreType.DMA,
        ],
        compiler_params=pltpu.CompilerParams(use_tc_tiling_on_sc=False),
    )

_EB = 16000
_RB = _EB // 8
_R8 = _E // 8


def _mlp_body(hs_ref, hd_ref, ef_ref, g_ref, w1g_ref, b1_ref,
              arep_ref, brep_ref, crep_ref, w2rep_ref, b2rep_ref,
              os_ref, od_ref):
    f32 = jnp.float32
    hs = hs_ref[...]
    hd = hd_ref[...]
    ef = ef_ref[...]
    g = g_ref[...]
    hs = jnp.where(jnp.isnan(hs), 0.0, hs)
    hd = jnp.where(jnp.isnan(hd), 0.0, hd)
    ef = jnp.where(jnp.isnan(ef), 0.0, ef)
    g = jnp.where(jnp.isnan(g), 0.0, g)
    const_row = jnp.dot(g, w1g_ref[...], preferred_element_type=f32) \
        + b1_ref[...]
    const_rep = jnp.concatenate([const_row] * 8, axis=1)
    hid = (jnp.dot(hs, arep_ref[...], preferred_element_type=f32)
           + jnp.dot(hd, brep_ref[...], preferred_element_type=f32)
           + jnp.dot(ef, crep_ref[...], preferred_element_type=f32)
           + const_rep)
    hid = jnp.tanh(hid)
    out = jnp.dot(hid, w2rep_ref[...], preferred_element_type=f32) \
        + b2rep_ref[...]
    out = jnp.tanh(out)
    os_ref[...] = out[:, 0:128]
    od_ref[...] = out[:, 128:256]


def _mlp(hs8, hd8, ef8, g, w1g, b1, arep, brep, crep, w2rep, b2rep):
    rows = hs8.shape[0]
    full = lambda shape: pl.BlockSpec(shape, lambda i: (0, 0))
    return pl.pallas_call(
        _mlp_body,
        grid=(rows // _RB,),
        in_specs=[
            pl.BlockSpec((_RB, 128), lambda i: (i, 0)),
            pl.BlockSpec((_RB, 128), lambda i: (i, 0)),
            pl.BlockSpec((_RB, 32), lambda i: (i, 0)),
            full((1, 11)),
            full((11, 64)),
            full((1, 64)),
            full((128, 512)),
            full((128, 512)),
            full((32, 512)),
            full((512, 256)),
            full((1, 256)),
        ],
        out_specs=[
            pl.BlockSpec((_RB, 128), lambda i: (i, 0)),
            pl.BlockSpec((_RB, 128), lambda i: (i, 0)),
        ],
        out_shape=[
            jax.ShapeDtypeStruct((rows, 128), jnp.float32),
            jax.ShapeDtypeStruct((rows, 128), jnp.float32),
        ],
    )(hs8, hd8, ef8, g, w1g, b1, arep, brep, crep, w2rep, b2rep)


def _block_diag(w, reps):
    k, m = w.shape
    out = jnp.zeros((reps * k, reps * m), w.dtype)
    for i in range(reps):
        out = out.at[i * k:(i + 1) * k, i * m:(i + 1) * m].set(w)
    return out


_NB = 5000


def _combine_body(a_ref, o_ref):
    a = a_ref[...]
    acc = a[0]
    for j in range(1, a.shape[0]):
        acc = acc + a[j]
    o_ref[...] = jnp.tanh(acc)


def _combine(acc):
    nparts = acc.shape[0]
    return pl.pallas_call(
        _combine_body,
        grid=(_N // _NB,),
        in_specs=[pl.BlockSpec((nparts, _NB, _H), lambda i: (0, i, 0))],
        out_specs=pl.BlockSpec((_NB, _H), lambda i: (i, 0)),
        out_shape=jax.ShapeDtypeStruct((_N, _H), jnp.float32),
    )(acc)


def kernel(h_local, h_global, addr_src, addr_dst, edge_feats, g_feat, t,
           W1_src, b1_src, W2_src, b2_src, W1_dst, b1_dst, W2_dst, b2_dst):
    a_src2 = addr_src.reshape(_NCHTOT * 8, _R8C)
    a_dst2 = addr_dst.reshape(_NCHTOT * 8, _R8C)

    gvec = jnp.concatenate([h_global, g_feat, t], axis=1)
    w1 = jnp.concatenate([W1_src, W1_dst], axis=1)
    b1 = jnp.concatenate([b1_src, b1_dst])[None, :]
    w1g = w1[2 * _H + _F:, :]
    arep = _block_diag(w1[0:_H, :], 8)
    brep = _block_diag(w1[_H:2 * _H, :], 8)
    crep = _block_diag(w1[2 * _H:2 * _H + _F, :], 8)
    w2rep = jnp.zeros((512, 256), jnp.float32)
    for e in range(8):
        w2rep = w2rep.at[e * 64:e * 64 + 32, e * 16:e * 16 + 16].set(W2_src)
        w2rep = w2rep.at[e * 64 + 32:e * 64 + 64,
                         128 + e * 16:128 + e * 16 + 16].set(W2_dst)
    b2rep = jnp.concatenate([jnp.tile(b2_src, 8), jnp.tile(b2_dst, 8)])[None, :]

    ef_perm = edge_feats.reshape(_NCHTOT, 8, _R8C, _F) \
        .transpose(0, 2, 1, 3).reshape(_R8, 8 * _F)
    zeros = jnp.zeros((_N, _H), jnp.float32)

    hs, hd = _gather()(h_local, a_src2, a_dst2)
    d_s, d_d = _mlp(hs, hd, ef_perm, gvec,
                    w1g, b1, arep, brep, crep, w2rep, b2rep)
    acc = _scatter()(zeros, a_src2, a_dst2, d_s, d_d)
    return _combine(acc)
